# baseline (speedup 1.0000x reference)
import jax
import jax.numpy as jnp
from jax.experimental import pallas as pl
from jax.experimental.pallas import tpu as pltpu

_MT = 400


def _hi_dot(x, w):
    return jax.lax.dot_general(
        x, w, (((1,), (0,)), ((), ())),
        precision=jax.lax.Precision.HIGHEST,
        preferred_element_type=jnp.float32)


def _s1_body(x_ref, w_ref, o_ref):
    o_ref[...] = _hi_dot(x_ref[...], w_ref[...]).astype(jnp.bfloat16)


def _layer1_body(adj_ref, s1_ref, b1_ref, w2_ref, o_ref):
    a = adj_ref[...].astype(jnp.bfloat16)
    acc = jnp.dot(a, s1_ref[...], preferred_element_type=jnp.float32)
    h = jnp.maximum(acc + b1_ref[...], 0.0)
    o_ref[...] = _hi_dot(h, w2_ref[...]).astype(jnp.bfloat16)


def _layer2_body(adj_ref, h2_ref, b2_ref, o_ref):
    a = adj_ref[...].astype(jnp.bfloat16)
    x = jnp.dot(a, h2_ref[...], preferred_element_type=jnp.float32) + b2_ref[...]
    m = jnp.max(x, axis=1, keepdims=True)
    s = x - m
    o_ref[...] = s - jnp.log(jnp.sum(jnp.exp(s), axis=1, keepdims=True))


def kernel(feature, adj, W1, b1, W2, b2):
    n, d_in = feature.shape
    d_hid = W1.shape[1]
    d_out = W2.shape[1]
    ni = n // _MT

    s1 = pl.pallas_call(
        _s1_body,
        out_shape=jax.ShapeDtypeStruct((n, d_hid), jnp.bfloat16),
    )(feature, W1)

    h2 = pl.pallas_call(
        _layer1_body,
        grid=(ni,),
        in_specs=[
            pl.BlockSpec((_MT, n), lambda i: (i, 0)),
            pl.BlockSpec((n, d_hid), lambda i: (0, 0)),
            pl.BlockSpec((1, d_hid), lambda i: (0, 0)),
            pl.BlockSpec((d_hid, d_out), lambda i: (0, 0)),
        ],
        out_specs=pl.BlockSpec((_MT, d_out), lambda i: (i, 0)),
        out_shape=jax.ShapeDtypeStruct((n, d_out), jnp.bfloat16),
        compiler_params=pltpu.CompilerParams(
            dimension_semantics=("arbitrary",)),
    )(adj, s1, b1.reshape(1, -1), W2)

    out = pl.pallas_call(
        _layer2_body,
        grid=(ni,),
        in_specs=[
            pl.BlockSpec((_MT, n), lambda i: (i, 0)),
            pl.BlockSpec((n, d_out), lambda i: (0, 0)),
            pl.BlockSpec((1, d_out), lambda i: (0, 0)),
        ],
        out_specs=pl.BlockSpec((_MT, d_out), lambda i: (i, 0)),
        out_shape=jax.ShapeDtypeStruct((n, d_out), jnp.float32),
        compiler_params=pltpu.CompilerParams(
            dimension_semantics=("arbitrary",)),
    )(adj, h2, b2.reshape(1, -1))

    return out
